# baseline (speedup 1.0000x reference)
import functools

import jax
import jax.numpy as jnp
from jax import lax
from jax.experimental import pallas as pl
from jax.experimental.pallas import tpu as pltpu
from jax.experimental.pallas import tpu_sc as plsc

N = 10000
NPAD = 10240
TRASH = N
E = 160000
EPAD = 163840
NROWS = EPAD // 128
NC, NS = 2, 16
RPT = NROWS // NS
EVAC = NPAD // NS

_mesh = plsc.VectorSubcoreMesh(
    core_axis_name="c", subcore_axis_name="s", num_cores=NC, num_subcores=NS
)


DEGW = 128


def _make_deg(w):
    @functools.partial(
        pl.kernel,
        out_type=jax.ShapeDtypeStruct((NPAD, w), jnp.float32),
        mesh=_mesh,
        scratch_types=[
            pltpu.VMEM((RPT, 128), jnp.int32),
            pltpu.VMEM((128, w), jnp.float32),
            pltpu.VMEM_SHARED((NPAD, w), jnp.float32),
            pltpu.SemaphoreType.DMA,
        ],
    )
    def _deg(dst_hbm, ones_hbm, zeros_hbm, deg_hbm, dst_v, ones_v, deg_sh, sem):
        c = lax.axis_index("c")
        s = lax.axis_index("s")
        pltpu.sync_copy(ones_hbm, ones_v)
        pltpu.sync_copy(dst_hbm.at[pl.ds(s * RPT, RPT)], dst_v)
        pltpu.sync_copy(zeros_hbm, deg_sh.at[pl.ds(s * EVAC, EVAC)])
        plsc.subcore_barrier()

        def body(j, carry):
            pltpu.async_copy(ones_v, deg_sh.at[dst_v.at[j]], sem, add=True)
            return carry

        lax.fori_loop(0, RPT, body, 0)

        def drain(j, carry):
            pltpu.make_async_copy(ones_hbm, ones_v, sem).wait()
            return carry

        lax.fori_loop(0, RPT, drain, 0)
        plsc.subcore_barrier()

        @pl.when(c == 0)
        def _():
            pltpu.sync_copy(
                deg_sh.at[pl.ds(s * EVAC, EVAC)], deg_hbm.at[pl.ds(s * EVAC, EVAC)]
            )

    return _deg


_deg_kernel = _make_deg(DEGW)


PH = RPT // 2


def _make_spmm(dh):
    @functools.partial(
        pl.kernel,
        out_type=jax.ShapeDtypeStruct((2 * NPAD, dh), jnp.float32),
        mesh=_mesh,
        scratch_types=[
            pltpu.VMEM((PH, 128), jnp.int32),
            pltpu.VMEM((PH, 128), jnp.int32),
            pltpu.VMEM((128, dh), jnp.float32),
            pltpu.VMEM((128, dh), jnp.float32),
            pltpu.VMEM_SHARED((NPAD, dh), jnp.float32),
            pltpu.SemaphoreType.DMA,
            pltpu.SemaphoreType.DMA,
            pltpu.SemaphoreType.DMA,
            pltpu.SemaphoreType.DMA,
        ],
    )
    def _spmm(
        z_hbm, src_hbm, dst_hbm, acc_hbm, src_v, dst_v, b0, b1, acc_sh, g0, g1, s0, s1
    ):
        c = lax.axis_index("c")
        s = lax.axis_index("s")
        pltpu.sync_copy(
            z_hbm.at[pl.ds(c * NPAD + s * EVAC, EVAC)],
            acc_sh.at[pl.ds(s * EVAC, EVAC)],
        )
        plsc.subcore_barrier()

        def wait_sem(buf, sem):
            pltpu.make_async_copy(z_hbm.at[pl.ds(0, 128)], buf, sem).wait()

        def phase(base):
            pltpu.sync_copy(src_hbm.at[c, pl.ds(s * RPT + base, PH)], src_v)
            pltpu.sync_copy(dst_hbm.at[pl.ds(s * RPT + base, PH)], dst_v)
            pltpu.async_copy(z_hbm.at[src_v.at[0]], b0, g0)
            pltpu.async_copy(z_hbm.at[src_v.at[1]], b1, g1)

            def body(i, carry):
                j0 = 2 * i
                wait_sem(b0, g0)
                pltpu.async_copy(b0, acc_sh.at[dst_v.at[j0]], s0, add=True)
                wait_sem(b1, g1)
                pltpu.async_copy(b1, acc_sh.at[dst_v.at[j0 + 1]], s1, add=True)

                @pl.when(i < PH // 2 - 1)
                def _():
                    wait_sem(b0, s0)
                    pltpu.async_copy(z_hbm.at[src_v.at[j0 + 2]], b0, g0)
                    wait_sem(b1, s1)
                    pltpu.async_copy(z_hbm.at[src_v.at[j0 + 3]], b1, g1)

                return carry

            lax.fori_loop(0, PH // 2, body, 0)
            wait_sem(b0, s0)
            wait_sem(b1, s1)

        phase(0)
        phase(PH)
        plsc.subcore_barrier()
        pltpu.sync_copy(
            acc_sh.at[pl.ds(s * EVAC, EVAC)],
            acc_hbm.at[pl.ds(c * NPAD + s * EVAC, EVAC)],
        )

    return _spmm


_spmm128 = _make_spmm(128)


def _tc1_body(deg_ref, x_ref, w1_ref, z_ref, dinv_ref):
    deg = jnp.sum(deg_ref[...], axis=1, keepdims=True) * (1.0 / DEGW) + 1.0
    dinv = 1.0 / jnp.sqrt(deg)
    dinv_ref[...] = dinv
    d10 = dinv[:N]
    z = jnp.dot(x_ref[...], w1_ref[...], preferred_element_type=jnp.float32) * d10
    z_ref[0, :N] = z[:, :128]
    z_ref[1, :N] = z[:, 128:]


_tc1 = pl.pallas_call(
    _tc1_body,
    out_shape=(
        jax.ShapeDtypeStruct((2, NPAD, 128), jnp.float32),
        jax.ShapeDtypeStruct((NPAD, 1), jnp.float32),
    ),
)


def _tc2_body(acc_ref, dinv_ref, b1_ref, w2_ref, z2_ref):
    d10 = dinv_ref[:N]
    accf = jnp.concatenate([acc_ref[0, :N], acc_ref[1, :N]], axis=1)
    h = jnp.maximum(accf * d10 + b1_ref[...][None, :], 0.0)
    z2 = jnp.dot(h, w2_ref[...], preferred_element_type=jnp.float32) * d10
    z2_ref[0, :N] = z2
    z2_ref[1, :N] = z2


_tc2 = pl.pallas_call(
    _tc2_body,
    out_shape=jax.ShapeDtypeStruct((2, NPAD, 128), jnp.float32),
)


def _tc3_body(acc_ref, dinv_ref, b2_ref, out_ref):
    d10 = dinv_ref[:N]
    out_ref[...] = acc_ref[0, :N] * d10 + b2_ref[...][None, :]


_tc3 = pl.pallas_call(
    _tc3_body,
    out_shape=jax.ShapeDtypeStruct((N, 128), jnp.float32),
)


def kernel(x, edge_index, W1, b1, W2, b2):
    src = edge_index[0]
    dst = edge_index[1]
    src_p = jnp.concatenate(
        [src, jnp.zeros((EPAD - E,), jnp.int32)]
    ).reshape(NROWS, 128)
    dst_p = jnp.concatenate(
        [dst, jnp.full((EPAD - E,), TRASH, jnp.int32)]
    ).reshape(NROWS, 128)
    src_both = jnp.stack([src_p, src_p + NPAD])
    ones8 = jnp.ones((128, DEGW), jnp.float32)
    zeros8 = jnp.zeros((EVAC, DEGW), jnp.float32)

    deg = _deg_kernel(dst_p, ones8, zeros8)
    z1p, dinv = _tc1(deg, x, W1)
    acc1 = _spmm128(z1p.reshape(2 * NPAD, 128), src_both, dst_p).reshape(2, NPAD, 128)
    z2p = _tc2(acc1, dinv, b1, W2)
    acc2 = _spmm128(z2p.reshape(2 * NPAD, 128), src_both, dst_p).reshape(2, NPAD, 128)
    return _tc3(acc2, dinv, b2)


# speedup vs baseline: 1.2476x; 1.2476x over previous
import functools

import jax
import jax.numpy as jnp
from jax import lax
from jax.experimental import pallas as pl
from jax.experimental.pallas import tpu as pltpu
from jax.experimental.pallas import tpu_sc as plsc

N = 10000
NPAD = 10240
TRASH = N
E = 160000
EPAD = 163840
NROWS = EPAD // 128
NC, NS = 2, 16
RPT = NROWS // NS
EVAC = NPAD // NS

_mesh = plsc.VectorSubcoreMesh(
    core_axis_name="c", subcore_axis_name="s", num_cores=NC, num_subcores=NS
)


DEGW = 128
RPTD = NROWS // (NC * NS)


@functools.partial(
    pl.kernel,
    out_type=jax.ShapeDtypeStruct((NC, NPAD, DEGW), jnp.float32),
    mesh=_mesh,
    scratch_types=[
        pltpu.VMEM((RPTD, 128), jnp.int32),
        pltpu.VMEM((128, DEGW), jnp.float32),
        pltpu.VMEM_SHARED((NPAD, DEGW), jnp.float32),
        pltpu.SemaphoreType.DMA,
    ],
)
def _deg_kernel(dst_hbm, ones_hbm, zeros_hbm, deg_hbm, dst_v, ones_v, deg_sh, sem):
    c = lax.axis_index("c")
    s = lax.axis_index("s")
    w = c * NS + s
    pltpu.sync_copy(ones_hbm, ones_v)
    pltpu.sync_copy(dst_hbm.at[pl.ds(w * RPTD, RPTD)], dst_v)
    pltpu.sync_copy(zeros_hbm, deg_sh.at[pl.ds(s * EVAC, EVAC)])
    plsc.subcore_barrier()

    def body(j, carry):
        pltpu.async_copy(ones_v, deg_sh.at[dst_v.at[j]], sem, add=True)
        return carry

    lax.fori_loop(0, RPTD, body, 0)

    def drain(j, carry):
        pltpu.make_async_copy(ones_hbm, ones_v, sem).wait()
        return carry

    lax.fori_loop(0, RPTD, drain, 0)
    plsc.subcore_barrier()
    pltpu.sync_copy(
        deg_sh.at[pl.ds(s * EVAC, EVAC)], deg_hbm.at[c, pl.ds(s * EVAC, EVAC)]
    )


PH = RPT // 2


def _make_spmm(dh):
    @functools.partial(
        pl.kernel,
        out_type=jax.ShapeDtypeStruct((2 * NPAD, dh), jnp.float32),
        mesh=_mesh,
        scratch_types=[
            pltpu.VMEM((PH, 128), jnp.int32),
            pltpu.VMEM((PH, 128), jnp.int32),
            pltpu.VMEM((128, dh), jnp.float32),
            pltpu.VMEM((128, dh), jnp.float32),
            pltpu.VMEM_SHARED((NPAD, dh), jnp.float32),
            pltpu.SemaphoreType.DMA,
            pltpu.SemaphoreType.DMA,
            pltpu.SemaphoreType.DMA,
            pltpu.SemaphoreType.DMA,
        ],
    )
    def _spmm(
        z_hbm, src_hbm, dst_hbm, acc_hbm, src_v, dst_v, b0, b1, acc_sh, g0, g1, s0, s1
    ):
        c = lax.axis_index("c")
        s = lax.axis_index("s")
        pltpu.sync_copy(
            z_hbm.at[pl.ds(c * NPAD + s * EVAC, EVAC)],
            acc_sh.at[pl.ds(s * EVAC, EVAC)],
        )
        plsc.subcore_barrier()

        def wait_sem(buf, sem):
            pltpu.make_async_copy(z_hbm.at[pl.ds(0, 128)], buf, sem).wait()

        def phase(base):
            pltpu.sync_copy(src_hbm.at[c, pl.ds(s * RPT + base, PH)], src_v)
            pltpu.sync_copy(dst_hbm.at[pl.ds(s * RPT + base, PH)], dst_v)
            pltpu.async_copy(z_hbm.at[src_v.at[0]], b0, g0)
            pltpu.async_copy(z_hbm.at[src_v.at[1]], b1, g1)

            def body(i, carry):
                j0 = 2 * i
                wait_sem(b0, g0)
                pltpu.async_copy(b0, acc_sh.at[dst_v.at[j0]], s0, add=True)
                wait_sem(b1, g1)
                pltpu.async_copy(b1, acc_sh.at[dst_v.at[j0 + 1]], s1, add=True)

                @pl.when(i < PH // 2 - 1)
                def _():
                    wait_sem(b0, s0)
                    pltpu.async_copy(z_hbm.at[src_v.at[j0 + 2]], b0, g0)
                    wait_sem(b1, s1)
                    pltpu.async_copy(z_hbm.at[src_v.at[j0 + 3]], b1, g1)

                return carry

            lax.fori_loop(0, PH // 2, body, 0)
            wait_sem(b0, s0)
            wait_sem(b1, s1)

        phase(0)
        phase(PH)
        plsc.subcore_barrier()
        pltpu.sync_copy(
            acc_sh.at[pl.ds(s * EVAC, EVAC)],
            acc_hbm.at[pl.ds(c * NPAD + s * EVAC, EVAC)],
        )

    return _spmm


_spmm128 = _make_spmm(128)


@functools.partial(
    pl.kernel,
    out_type=jax.ShapeDtypeStruct((2 * NPAD, 128), jnp.float32),
    mesh=_mesh,
    scratch_types=[
        pltpu.VMEM((RPTD, 128), jnp.int32),
        pltpu.VMEM((RPTD, 128), jnp.int32),
        pltpu.VMEM((128, 128), jnp.float32),
        pltpu.VMEM((128, 128), jnp.float32),
        pltpu.VMEM_SHARED((NPAD, 128), jnp.float32),
        pltpu.SemaphoreType.DMA,
        pltpu.SemaphoreType.DMA,
        pltpu.SemaphoreType.DMA,
        pltpu.SemaphoreType.DMA,
    ],
)
def _spmm_split(
    z_hbm, src_hbm, dst_hbm, zeros_hbm, acc_hbm,
    src_v, dst_v, b0, b1, acc_sh, g0, g1, s0, s1,
):
    c = lax.axis_index("c")
    s = lax.axis_index("s")
    w = c * NS + s

    @pl.when(c == 0)
    def _():
        pltpu.sync_copy(
            z_hbm.at[pl.ds(s * EVAC, EVAC)], acc_sh.at[pl.ds(s * EVAC, EVAC)]
        )

    @pl.when(c == 1)
    def _():
        pltpu.sync_copy(zeros_hbm, acc_sh.at[pl.ds(s * EVAC, EVAC)])

    pltpu.sync_copy(src_hbm.at[pl.ds(w * RPTD, RPTD)], src_v)
    pltpu.sync_copy(dst_hbm.at[pl.ds(w * RPTD, RPTD)], dst_v)
    plsc.subcore_barrier()

    def wait_sem(buf, sem):
        pltpu.make_async_copy(z_hbm.at[pl.ds(0, 128)], buf, sem).wait()

    pltpu.async_copy(z_hbm.at[src_v.at[0]], b0, g0)
    pltpu.async_copy(z_hbm.at[src_v.at[1]], b1, g1)

    def body(i, carry):
        j0 = 2 * i
        wait_sem(b0, g0)
        pltpu.async_copy(b0, acc_sh.at[dst_v.at[j0]], s0, add=True)
        wait_sem(b1, g1)
        pltpu.async_copy(b1, acc_sh.at[dst_v.at[j0 + 1]], s1, add=True)

        @pl.when(i < RPTD // 2 - 1)
        def _():
            wait_sem(b0, s0)
            pltpu.async_copy(z_hbm.at[src_v.at[j0 + 2]], b0, g0)
            wait_sem(b1, s1)
            pltpu.async_copy(z_hbm.at[src_v.at[j0 + 3]], b1, g1)

        return carry

    lax.fori_loop(0, RPTD // 2, body, 0)
    wait_sem(b0, s0)
    wait_sem(b1, s1)
    plsc.subcore_barrier()
    pltpu.sync_copy(
        acc_sh.at[pl.ds(s * EVAC, EVAC)],
        acc_hbm.at[pl.ds(c * NPAD + s * EVAC, EVAC)],
    )


def _tc1_body(deg_ref, x_ref, w1_ref, z_ref, dinv_ref):
    dsum = jnp.sum(deg_ref[0], axis=1, keepdims=True) + jnp.sum(
        deg_ref[1], axis=1, keepdims=True
    )
    deg = dsum * (1.0 / DEGW) + 1.0
    dinv = 1.0 / jnp.sqrt(deg)
    dinv_ref[...] = dinv
    d10 = dinv[:N]
    z = jnp.dot(x_ref[...], w1_ref[...], preferred_element_type=jnp.float32) * d10
    z_ref[0, :N] = z[:, :128]
    z_ref[1, :N] = z[:, 128:]


_tc1 = pl.pallas_call(
    _tc1_body,
    out_shape=(
        jax.ShapeDtypeStruct((2, NPAD, 128), jnp.float32),
        jax.ShapeDtypeStruct((NPAD, 1), jnp.float32),
    ),
)


def _tc2_body(acc_ref, dinv_ref, b1_ref, w2_ref, z2_ref):
    d10 = dinv_ref[:N]
    accf = jnp.concatenate([acc_ref[0, :N], acc_ref[1, :N]], axis=1)
    h = jnp.maximum(accf * d10 + b1_ref[...][None, :], 0.0)
    z2 = jnp.dot(h, w2_ref[...], preferred_element_type=jnp.float32) * d10
    z2_ref[:N] = z2


_tc2 = pl.pallas_call(
    _tc2_body,
    out_shape=jax.ShapeDtypeStruct((NPAD, 128), jnp.float32),
)


def _tc3_body(acc_ref, dinv_ref, b2_ref, out_ref):
    d10 = dinv_ref[:N]
    out_ref[...] = (acc_ref[0, :N] + acc_ref[1, :N]) * d10 + b2_ref[...][None, :]


_tc3 = pl.pallas_call(
    _tc3_body,
    out_shape=jax.ShapeDtypeStruct((N, 128), jnp.float32),
)


def kernel(x, edge_index, W1, b1, W2, b2):
    src = edge_index[0]
    dst = edge_index[1]
    src_p = jnp.concatenate(
        [src, jnp.zeros((EPAD - E,), jnp.int32)]
    ).reshape(NROWS, 128)
    dst_p = jnp.concatenate(
        [dst, jnp.full((EPAD - E,), TRASH, jnp.int32)]
    ).reshape(NROWS, 128)
    src_both = jnp.stack([src_p, src_p + NPAD])
    ones128 = jnp.ones((128, DEGW), jnp.float32)
    zeros128 = jnp.zeros((EVAC, DEGW), jnp.float32)

    deg = _deg_kernel(dst_p, ones128, zeros128)
    z1p, dinv = _tc1(deg, x, W1)
    acc1 = _spmm128(z1p.reshape(2 * NPAD, 128), src_both, dst_p).reshape(2, NPAD, 128)
    z2p = _tc2(acc1, dinv, b1, W2)
    acc2 = _spmm_split(z2p, src_p, dst_p, zeros128).reshape(2, NPAD, 128)
    return _tc3(acc2, dinv, b2)
